# baseline (speedup 1.0000x reference)
import functools

import jax
import jax.numpy as jnp
from jax import lax
from jax.experimental import pallas as pl
from jax.experimental.pallas import tpu as pltpu
from jax.experimental.pallas import tpu_sc as plsc

B = 4096
L = 200
NC = 2
NS = 16
NW = NC * NS
CPW = B // NW
LANES = 16
NG = CPW // LANES

_mesh = plsc.VectorSubcoreMesh(core_axis_name="c", subcore_axis_name="s")


@functools.partial(
    pl.kernel,
    out_type=jax.ShapeDtypeStruct((NW, LANES), jnp.float32),
    mesh=_mesh,
    scratch_types=[
        pltpu.VMEM((L, CPW), jnp.float32),
        pltpu.VMEM((CPW,), jnp.int32),
        pltpu.VMEM((CPW,), jnp.int32),
        pltpu.VMEM((CPW,), jnp.float32),
        pltpu.VMEM((LANES,), jnp.float32),
        pltpu.SemaphoreType.DMA,
        pltpu.SemaphoreType.DMA,
        pltpu.SemaphoreType.DMA,
        pltpu.SemaphoreType.DMA,
    ],
    compiler_params=pltpu.CompilerParams(needs_layout_passes=False),
)
def _sc_partials(xt_hbm, il_hbm, un_hbm, y_hbm, out_hbm,
                 xv, ilv, unv, yv, stage, sem0, sem1, sem2, sem3):
    cid = lax.axis_index("c")
    sid = lax.axis_index("s")
    wid = sid * NC + cid
    cbase = wid * CPW

    c0 = pltpu.async_copy(xt_hbm.at[:, pl.ds(cbase, CPW)], xv, sem0)
    c1 = pltpu.async_copy(il_hbm.at[pl.ds(cbase, CPW)], ilv, sem1)
    c2 = pltpu.async_copy(un_hbm.at[pl.ds(cbase, CPW)], unv, sem2)
    c3 = pltpu.async_copy(y_hbm.at[pl.ds(cbase, CPW)], yv, sem3)
    c1.wait()
    c2.wait()
    c3.wait()
    c0.wait()

    iota = lax.iota(jnp.int32, LANES)
    hps = [jnp.where(unv[pl.ds(g * LANES, LANES)] != 0,
                     ilv[pl.ds(g * LANES, LANES)] >> 1, 0)
           for g in range(NG)]
    colvecs = [iota + g * LANES for g in range(NG)]
    pairs_vec = jnp.zeros((LANES,), jnp.int32)
    for g in range(NG):
        pairs_vec = pairs_vec + hps[g]

    HMAX = (L - 1) // 2

    def body_a(i, carry):
        sq, rss = carry
        ivec = jnp.full((LANES,), i, dtype=jnp.int32)
        rss_new = []
        for g in range(NG):
            f = xv[i, pl.ds(g * LANES, LANES)]
            rss_new.append(rss[g] + f)
            pm = ivec < hps[g]
            s = plsc.load_gather(xv, [ivec + hps[g], colvecs[g]], mask=pm)
            d = f - s
            sq = sq + jnp.where(pm, d * d, 0.0)
        return sq, tuple(rss_new)

    def body_b(i, rss):
        return tuple(rss[g] + xv[i, pl.ds(g * LANES, LANES)]
                     for g in range(NG))

    zero = jnp.zeros((LANES,), jnp.float32)
    sq_acc, rss = lax.fori_loop(0, HMAX, body_a,
                                (zero, (zero,) * NG), unroll=3)
    rss = lax.fori_loop(HMAX, L, body_b, rss, unroll=4)

    loss_acc = jnp.zeros((LANES,), jnp.float32)
    for g in range(NG):
        dv = rss[g] - yv[pl.ds(g * LANES, LANES)]
        loss_acc = loss_acc + dv * dv

    sq_s = jnp.sum(sq_acc)
    pair_s = jnp.sum(pairs_vec).astype(jnp.float32)
    loss_s = jnp.sum(loss_acc)
    v = jnp.where(iota == 0, sq_s,
                  jnp.where(iota == 1, pair_s,
                            jnp.where(iota == 2, loss_s, 0.0)))
    stage[...] = v
    pltpu.sync_copy(stage, out_hbm.at[wid])


def _combine_body(p_ref, o_ref):
    p = p_ref[...]
    sq = jnp.sum(p[:, 0])
    pr = jnp.sum(p[:, 1])
    ls = jnp.sum(p[:, 2])
    total = ls / jnp.float32(B) + sq / jnp.maximum(pr, 1.0)
    o_ref[...] = jnp.full((1, 1), total, dtype=jnp.float32)


def kernel(outputs, y, unrolled, inst_len):
    un = unrolled.astype(jnp.int32)
    part = _sc_partials(outputs.T, inst_len.astype(jnp.int32), un, y)
    total = pl.pallas_call(
        _combine_body,
        out_shape=jax.ShapeDtypeStruct((1, 1), jnp.float32),
    )(part)
    return total[0, 0]


# speedup vs baseline: 1.0113x; 1.0113x over previous
import functools

import jax
import jax.numpy as jnp
from jax import lax
from jax.experimental import pallas as pl
from jax.experimental.pallas import tpu as pltpu
from jax.experimental.pallas import tpu_sc as plsc

B = 4096
L = 200
NC = 2
NS = 16
NW = NC * NS
CPW = B // NW
LANES = 16
NG = CPW // LANES

_mesh = plsc.VectorSubcoreMesh(core_axis_name="c", subcore_axis_name="s")


@functools.partial(
    pl.kernel,
    out_type=jax.ShapeDtypeStruct((NW, LANES), jnp.float32),
    mesh=_mesh,
    scratch_types=[
        pltpu.VMEM((L, CPW), jnp.float32),
        pltpu.VMEM((CPW,), jnp.int32),
        pltpu.VMEM((CPW,), jnp.int32),
        pltpu.VMEM((CPW,), jnp.float32),
        pltpu.VMEM((LANES,), jnp.float32),
        pltpu.SemaphoreType.DMA,
        pltpu.SemaphoreType.DMA,
        pltpu.SemaphoreType.DMA,
        pltpu.SemaphoreType.DMA,
    ],
    compiler_params=pltpu.CompilerParams(needs_layout_passes=False),
)
def _sc_partials(xt_hbm, il_hbm, un_hbm, y_hbm, out_hbm,
                 xv, ilv, unv, yv, stage, sem0, sem1, sem2, sem3):
    cid = lax.axis_index("c")
    sid = lax.axis_index("s")
    wid = sid * NC + cid
    cbase = wid * CPW

    c0 = pltpu.async_copy(xt_hbm.at[:, pl.ds(cbase, CPW)], xv, sem0)
    c1 = pltpu.async_copy(il_hbm.at[pl.ds(cbase, CPW)], ilv, sem1)
    c2 = pltpu.async_copy(un_hbm.at[pl.ds(cbase, CPW)], unv, sem2)
    c3 = pltpu.async_copy(y_hbm.at[pl.ds(cbase, CPW)], yv, sem3)
    c1.wait()
    c2.wait()
    c3.wait()
    c0.wait()

    iota = lax.iota(jnp.int32, LANES)
    hps = [jnp.where(unv[pl.ds(g * LANES, LANES)] != 0,
                     ilv[pl.ds(g * LANES, LANES)] >> 1, 0)
           for g in range(NG)]
    colvecs = [iota + g * LANES for g in range(NG)]
    pairs_vec = jnp.zeros((LANES,), jnp.int32)
    for g in range(NG):
        pairs_vec = pairs_vec + hps[g]

    HMAX = (L - 1) // 2

    def body_a(i, carry):
        sq, rss = carry
        ivec = jnp.full((LANES,), i, dtype=jnp.int32)
        rss_new = []
        for g in range(NG):
            f = xv[i, pl.ds(g * LANES, LANES)]
            rss_new.append(rss[g] + f)
            pm = ivec < hps[g]
            s = plsc.load_gather(xv, [ivec + hps[g], colvecs[g]], mask=pm)
            d = f - s
            sq = sq + jnp.where(pm, d * d, 0.0)
        return sq, tuple(rss_new)

    def body_b(i, rss):
        return tuple(rss[g] + xv[i, pl.ds(g * LANES, LANES)]
                     for g in range(NG))

    zero = jnp.zeros((LANES,), jnp.float32)
    sq_acc, rss = lax.fori_loop(0, HMAX, body_a,
                                (zero, (zero,) * NG))
    rss = lax.fori_loop(HMAX, L, body_b, rss)

    loss_acc = jnp.zeros((LANES,), jnp.float32)
    for g in range(NG):
        dv = rss[g] - yv[pl.ds(g * LANES, LANES)]
        loss_acc = loss_acc + dv * dv

    sq_s = jnp.sum(sq_acc)
    pair_s = jnp.sum(pairs_vec).astype(jnp.float32)
    loss_s = jnp.sum(loss_acc)
    v = jnp.where(iota == 0, sq_s,
                  jnp.where(iota == 1, pair_s,
                            jnp.where(iota == 2, loss_s, 0.0)))
    stage[...] = v
    pltpu.sync_copy(stage, out_hbm.at[wid])


def _combine_body(p_ref, o_ref):
    p = p_ref[...]
    sq = jnp.sum(p[:, 0])
    pr = jnp.sum(p[:, 1])
    ls = jnp.sum(p[:, 2])
    total = ls / jnp.float32(B) + sq / jnp.maximum(pr, 1.0)
    o_ref[...] = jnp.full((1, 1), total, dtype=jnp.float32)


def kernel(outputs, y, unrolled, inst_len):
    un = unrolled.astype(jnp.int32)
    part = _sc_partials(outputs.T, inst_len.astype(jnp.int32), un, y)
    total = pl.pallas_call(
        _combine_body,
        out_shape=jax.ShapeDtypeStruct((1, 1), jnp.float32),
    )(part)
    return total[0, 0]


# speedup vs baseline: 1.0152x; 1.0039x over previous
import functools

import jax
import jax.numpy as jnp
from jax import lax
from jax.experimental import pallas as pl
from jax.experimental.pallas import tpu as pltpu
from jax.experimental.pallas import tpu_sc as plsc

B = 4096
L = 200
NC = 2
NS = 16
NW = NC * NS
CPW = B // NW
LANES = 16
NG = CPW // LANES

_mesh = plsc.VectorSubcoreMesh(core_axis_name="c", subcore_axis_name="s")


@functools.partial(
    pl.kernel,
    out_type=jax.ShapeDtypeStruct((NW, LANES), jnp.float32),
    mesh=_mesh,
    scratch_types=[
        pltpu.VMEM((L, CPW), jnp.float32),
        pltpu.VMEM((CPW,), jnp.int32),
        pltpu.VMEM((CPW,), jnp.int32),
        pltpu.VMEM((CPW,), jnp.float32),
        pltpu.VMEM((LANES,), jnp.float32),
        pltpu.SemaphoreType.DMA,
        pltpu.SemaphoreType.DMA,
        pltpu.SemaphoreType.DMA,
        pltpu.SemaphoreType.DMA,
    ],
    compiler_params=pltpu.CompilerParams(needs_layout_passes=False),
)
def _sc_partials(xt_hbm, il_hbm, un_hbm, y_hbm, out_hbm,
                 xv, ilv, unv, yv, stage, sem0, sem1, sem2, sem3):
    cid = lax.axis_index("c")
    sid = lax.axis_index("s")
    wid = sid * NC + cid
    cbase = wid * CPW

    c0 = pltpu.async_copy(xt_hbm.at[:, pl.ds(cbase, CPW)], xv, sem0)
    c1 = pltpu.async_copy(il_hbm.at[pl.ds(cbase, CPW)], ilv, sem1)
    c2 = pltpu.async_copy(un_hbm.at[pl.ds(cbase, CPW)], unv, sem2)
    c3 = pltpu.async_copy(y_hbm.at[pl.ds(cbase, CPW)], yv, sem3)
    c1.wait()
    c2.wait()
    c3.wait()
    c0.wait()

    iota = lax.iota(jnp.int32, LANES)
    hps = [(ilv[pl.ds(g * LANES, LANES)] >> 1) * unv[pl.ds(g * LANES, LANES)]
           for g in range(NG)]
    colvecs = [iota + g * LANES for g in range(NG)]
    pairs_vec = jnp.zeros((LANES,), jnp.int32)
    for g in range(NG):
        pairs_vec = pairs_vec + hps[g]

    HMAX = (L - 1) // 2

    def body_a(i, carry):
        sq, rss = carry
        ivec = jnp.full((LANES,), i, dtype=jnp.int32)
        rss_new = []
        for g in range(NG):
            f = xv[i, pl.ds(g * LANES, LANES)]
            rss_new.append(rss[g] + f)
            pm = ivec < hps[g]
            s = plsc.load_gather(xv, [ivec + hps[g], colvecs[g]], mask=pm)
            d = f - s
            sq = sq + jnp.where(pm, d * d, 0.0)
        return sq, tuple(rss_new)

    def body_b(i, rss):
        return tuple(rss[g] + xv[i, pl.ds(g * LANES, LANES)]
                     for g in range(NG))

    zero = jnp.zeros((LANES,), jnp.float32)
    sq_acc, rss = lax.fori_loop(0, HMAX, body_a,
                                (zero, (zero,) * NG))
    rss = lax.fori_loop(HMAX, L, body_b, rss)

    loss_acc = jnp.zeros((LANES,), jnp.float32)
    for g in range(NG):
        dv = rss[g] - yv[pl.ds(g * LANES, LANES)]
        loss_acc = loss_acc + dv * dv

    sq_s = jnp.sum(sq_acc)
    pair_s = jnp.sum(pairs_vec).astype(jnp.float32)
    loss_s = jnp.sum(loss_acc)
    v = jnp.where(iota == 0, sq_s,
                  jnp.where(iota == 1, pair_s,
                            jnp.where(iota == 2, loss_s, 0.0)))
    stage[...] = v
    pltpu.sync_copy(stage, out_hbm.at[wid])


def _combine_body(p_ref, o_ref):
    p = p_ref[...]
    sq = jnp.sum(p[:, 0])
    pr = jnp.sum(p[:, 1])
    ls = jnp.sum(p[:, 2])
    total = ls / jnp.float32(B) + sq / jnp.maximum(pr, 1.0)
    o_ref[...] = jnp.full((1, 1), total, dtype=jnp.float32)


def kernel(outputs, y, unrolled, inst_len):
    un = unrolled.astype(jnp.int32)
    part = _sc_partials(outputs.T, inst_len.astype(jnp.int32), un, y)
    total = pl.pallas_call(
        _combine_body,
        out_shape=jax.ShapeDtypeStruct((1, 1), jnp.float32),
    )(part)
    return total[0, 0]
